# baseline (speedup 1.0000x reference)
import functools

import jax
import jax.numpy as jnp
import numpy as np
from jax import lax
from jax.experimental import pallas as pl
from jax.experimental.pallas import tpu as pltpu
from jax.experimental.pallas import tpu_sc as plsc

B, Q, D = 2, 900, 256
G, P, L, N = 4, 8, 4, 6
C = 64
IMG_H, IMG_W = 256, 704
PC = (-51.2, -51.2, -5.0, 51.2, 51.2, 3.0)
EPS = 1e-5
FEAT_SHAPES = [(32, 88), (16, 44), (8, 22), (4, 11)]

GP = G * P
TAPS = L * 4
NPTS = B * Q * GP
ROWS = B * Q
NW = 32
ITERS = (ROWS + NW - 1) // NW

_LEVEL_ROWS = [B * G * N * h * w for (h, w) in FEAT_SHAPES]
_LEVEL_BASE = [int(x) for x in np.cumsum([0] + _LEVEL_ROWS)[:4]]
TABLE_ROWS = int(np.sum(_LEVEL_ROWS))

_PERM_OFF = np.zeros((G * P * 3,), np.int32)
for _g in range(G):
    for _p in range(P):
        for _c in range(3):
            _PERM_OFF[_c * 32 + _g * 8 + _p] = _g * 24 + _p * 3 + _c
_PERM_SW = np.zeros((G * P * L,), np.int32)
for _g in range(G):
    for _p in range(P):
        for _l in range(L):
            _PERM_SW[_l * 32 + _g * 8 + _p] = _g * 32 + _p * 4 + _l

_PERM_CH = np.zeros((C,), np.int32)
for _h in range(2):
    for _i in range(16):
        for _k in range(2):
            _PERM_CH[_h * 32 + 2 * _i + _k] = _h * 32 + 16 * _k + _i

_SPREAD_MASK = [2 ** int(np.floor(np.log2(B * G * N * h * w))) - 1
                for (h, w) in FEAT_SHAPES]


def _tc_body(q_ref, spc_ref, ws_ref, bs_ref,
             idx_ref, w_ref, uv_ref):
    b = pl.program_id(0)
    q = q_ref[0]
    swl = jnp.dot(q, ws_ref[...], preferred_element_type=jnp.float32) + bs_ref[0]

    s = [swl[:, l * 32:(l + 1) * 32] for l in range(L)]
    m = jnp.maximum(jnp.maximum(s[0], s[1]), jnp.maximum(s[2], s[3]))
    e = [jnp.exp(sl - m) for sl in s]
    den = e[0] + e[1] + e[2] + e[3]
    sw = [el / den for el in e]

    best_v = jnp.full((Q, GP), -1.0, jnp.float32)
    best_i = jnp.zeros((Q, GP), jnp.int32)
    u_sel = jnp.zeros((Q, GP), jnp.float32)
    v_sel = jnp.zeros((Q, GP), jnp.float32)
    for n in range(N):
        sx = spc_ref[0, n * 3 + 0]
        sy = spc_ref[0, n * 3 + 1]
        hom = spc_ref[0, n * 3 + 2]
        homnz = jnp.maximum(hom, EPS)
        un = (sx / homnz) / IMG_W
        vn = (sy / homnz) / IMG_H
        val = ((hom > EPS) & (vn > 0.0) & (vn < 1.0)
               & (un > 0.0) & (un < 1.0)).astype(jnp.float32)
        upd = val > best_v
        best_i = jnp.where(upd, n, best_i)
        u_sel = jnp.where(upd, un, u_sel)
        v_sel = jnp.where(upd, vn, v_sel)
        best_v = jnp.maximum(best_v, val)

    uv_ref[0, :, 0:32] = u_sel
    uv_ref[0, :, 32:64] = v_sel

    gcol = lax.broadcasted_iota(jnp.int32, (Q, GP), 1) // P
    bg6 = (b * G + gcol) * N + best_i
    spread = (lax.broadcasted_iota(jnp.int32, (Q, GP), 0) * GP
              + lax.broadcasted_iota(jnp.int32, (Q, GP), 1))

    for l, (H, W) in enumerate(FEAT_SHAPES):
        xl = u_sel * W - 0.5
        yl = v_sel * H - 0.5
        x0f = jnp.floor(xl)
        y0f = jnp.floor(yl)
        x0 = x0f.astype(jnp.int32)
        y0 = y0f.astype(jnp.int32)
        wx = xl - x0f
        wy = yl - y0f
        rowbase = bg6 * (H * W)
        corners = [
            (0, 0, (1.0 - wx) * (1.0 - wy)),
            (0, 1, wx * (1.0 - wy)),
            (1, 0, (1.0 - wx) * wy),
            (1, 1, wx * wy),
        ]
        for ci, (dy, dx, cw) in enumerate(corners):
            xx = x0 + dx
            yy = y0 + dy
            ok = ((xx >= 0) & (xx < W) & (yy >= 0) & (yy < H)).astype(jnp.float32)
            xc = jnp.clip(xx, 0, W - 1)
            yc = jnp.clip(yy, 0, H - 1)
            t = l * 4 + ci
            wt = cw * ok * sw[l]
            row = rowbase + yc * W + xc
            row = jnp.where(wt > 0.0,
                            row, (spread * TAPS + t) & _SPREAD_MASK[l])
            idx_ref[0, :, t * 32:(t + 1) * 32] = row
            w_ref[0, :, t * 32:(t + 1) * 32] = wt


def _tc_precompute(query, spc, ws, bs):
    return pl.pallas_call(
        _tc_body,
        grid=(B,),
        in_specs=[
            pl.BlockSpec((1, Q, D), lambda b: (b, 0, 0)),
            pl.BlockSpec((1, 3 * N, Q, GP), lambda b: (b, 0, 0, 0)),
            pl.BlockSpec((D, 128), lambda b: (0, 0)),
            pl.BlockSpec((1, 128), lambda b: (0, 0)),
        ],
        out_specs=[
            pl.BlockSpec((1, Q, 512), lambda b: (b, 0, 0)),
            pl.BlockSpec((1, Q, 512), lambda b: (b, 0, 0)),
            pl.BlockSpec((1, Q, 64), lambda b: (b, 0, 0)),
        ],
        out_shape=[
            jax.ShapeDtypeStruct((B, Q, 512), jnp.int32),
            jax.ShapeDtypeStruct((B, Q, 512), jnp.float32),
            jax.ShapeDtypeStruct((B, Q, 64), jnp.float32),
        ],
    )(query, spc, ws, bs)


def _sc_gather(tables, idx, w):
    mesh = plsc.VectorSubcoreMesh(core_axis_name="c", subcore_axis_name="s")
    RPW = ITERS

    @functools.partial(
        pl.kernel, mesh=mesh,
        out_type=jax.ShapeDtypeStruct((NPTS // 2, 2 * C), jnp.float32),
        compiler_params=pltpu.CompilerParams(use_tc_tiling_on_sc=False,
                                             needs_layout_passes=False),
        scratch_types=[
            pltpu.VMEM((4 * RPW, 128), jnp.int32),
            pltpu.VMEM((4 * RPW, 128), jnp.float32),
            pltpu.VMEM((2, 512, C), jnp.bfloat16),
            pltpu.VMEM((2, GP // 2, 2 * C), jnp.float32),
            pltpu.SemaphoreType.DMA,
            pltpu.SemaphoreType.DMA,
            pltpu.SemaphoreType.DMA,
            pltpu.SemaphoreType.DMA,
        ],
    )
    def k(t0, t1, t2, t3, idx_hbm, w_hbm, out_hbm,
          idx_v, w_v, rows_v, out_v, g0, g1, so0, so1):
        tabs = (t0, t1, t2, t3)
        gsem = (g0, g1)
        osem = (so0, so1)
        wid = lax.axis_index("s") * 2 + lax.axis_index("c")
        nrows = jnp.where(wid < 8, RPW, RPW - 1)
        base = 56 * wid + jnp.minimum(wid, 8)

        @pl.when(wid < 8)
        def _():
            pltpu.sync_copy(idx_hbm.at[pl.ds(4 * base, 4 * RPW)], idx_v)
            pltpu.sync_copy(w_hbm.at[pl.ds(4 * base, 4 * RPW)], w_v)

        @pl.when(wid >= 8)
        def _():
            pltpu.sync_copy(idx_hbm.at[pl.ds(4 * base, 4 * (RPW - 1))],
                            idx_v.at[pl.ds(0, 4 * (RPW - 1))])
            pltpu.sync_copy(w_hbm.at[pl.ds(4 * base, 4 * (RPW - 1))],
                            w_v.at[pl.ds(0, 4 * (RPW - 1))])

        def fire(j, buf):
            @pl.when(j < nrows)
            def _():
                for kk in range(4):
                    pltpu.async_copy(
                        tabs[kk].at[idx_v.at[4 * j + kk]],
                        rows_v.at[buf].at[pl.ds(kk * 128, 128)], gsem[buf])

        def consume(j, buf):
            @pl.when(j < nrows)
            def _():
                pltpu.make_async_copy(
                    t0.at[pl.ds(0, 512)], rows_v.at[buf], gsem[buf]).wait()

                @pl.when(j >= 2)
                def _():
                    pltpu.make_async_copy(
                        out_v.at[buf], out_hbm.at[pl.ds(0, GP // 2)],
                        osem[buf]).wait()

                def pbody(p, c2):
                    wvec = w_v[4 * j + (p >> 3), pl.ds((p & 7) * TAPS, TAPS)]
                    wts = [wvec[t] for t in range(TAPS)]
                    terms = [[], [], [], []]
                    for t in range(TAPS):
                        for h in range(2):
                            xb = rows_v[buf, t * 32 + p, pl.ds(h * 32, 32)]
                            a0, a1 = plsc.unpack(
                                xb, format=plsc.PackFormat.INTERLEAVED)
                            terms[2 * h].append(a0 * wts[t])
                            terms[2 * h + 1].append(a1 * wts[t])
                    for c in range(4):
                        ts = terms[c]
                        while len(ts) > 1:
                            ts = [ts[a] + ts[a + 1]
                                  for a in range(0, len(ts), 2)]
                        out_v[buf, p >> 1,
                              pl.ds((p & 1) * C + c * 16, 16)] = ts[0]
                    return c2

                lax.fori_loop(0, GP, pbody, 0)
                pltpu.async_copy(out_v.at[buf],
                                 out_hbm.at[pl.ds((base + j) * (GP // 2),
                                                  GP // 2)],
                                 osem[buf])

        fire(0, 0)

        def body(i2, carry):
            j0 = 2 * i2
            fire(j0 + 1, 1)
            consume(j0, 0)
            fire(j0 + 2, 0)
            consume(j0 + 1, 1)
            return carry

        lax.fori_loop(0, (RPW + 1) // 2, body, 0)

        for buf in range(2):
            pltpu.make_async_copy(
                out_v.at[buf], out_hbm.at[pl.ds(0, GP // 2)],
                osem[buf]).wait()

    return k(tables[0], tables[1], tables[2], tables[3], idx, w)


def kernel(query, reference_points, lidar2img, feat0, feat1, feat2, feat3,
           W_off, b_off, W_sw, b_sw):
    perm_sw = jnp.asarray(_PERM_SW)
    ws = W_sw[:, perm_sw]
    bs = b_sw[perm_sw][None]

    so = (query @ W_off + b_off).reshape(B, Q, G, P, 3)
    rp = reference_points[:, :, None, None, :]
    rx = rp[..., 0:1] * (PC[3] - PC[0]) + PC[0]
    ry = rp[..., 1:2] * (PC[4] - PC[1]) + PC[1]
    rz = rp[..., 2:3] * (PC[5] - PC[2]) + PC[2]
    ref = jnp.concatenate([rx, ry, rz], axis=-1) + so
    sp = ref.reshape(B, Q, GP, 3)
    ones = jnp.ones_like(sp[..., :1])
    sph = jnp.concatenate([sp, ones], axis=-1)
    spc = jnp.einsum('bnij,bqpj->bnqpi', lidar2img, sph)
    spc_k = jnp.transpose(spc[..., :3], (0, 1, 4, 2, 3)).reshape(B, 3 * N, Q, GP)

    idx, w, uv = _tc_precompute(query, spc_k, ws, bs)

    perm_ch = jnp.asarray(_PERM_CH)
    tables = [jnp.transpose(f, (0, 2, 3, 4, 1)).reshape(-1, C)
              .astype(jnp.bfloat16)[:, perm_ch]
              for f in (feat0, feat1, feat2, feat3)]

    w_pm = jnp.transpose(w.reshape(B, Q, TAPS, GP),
                         (0, 1, 3, 2)).reshape(4 * ROWS, 128)
    out = _sc_gather(tables, idx.reshape(4 * ROWS, 128), w_pm)
    final = out.reshape(B, Q, G, P, C)

    u = uv[:, :, 0:32].reshape(B, Q, G, P)
    v = uv[:, :, 32:64].reshape(B, Q, G, P)
    ref2d = jnp.stack([u, v], axis=-1)
    ref2d = jnp.transpose(ref2d, (0, 2, 1, 3, 4)).reshape(B * G, Q, P, 2)
    return final, ref2d


# speedup vs baseline: 1.0856x; 1.0856x over previous
import functools

import jax
import jax.numpy as jnp
import numpy as np
from jax import lax
from jax.experimental import pallas as pl
from jax.experimental.pallas import tpu as pltpu
from jax.experimental.pallas import tpu_sc as plsc

B, Q, D = 2, 900, 256
G, P, L, N = 4, 8, 4, 6
C = 64
IMG_H, IMG_W = 256, 704
PC = (-51.2, -51.2, -5.0, 51.2, 51.2, 3.0)
EPS = 1e-5
FEAT_SHAPES = [(32, 88), (16, 44), (8, 22), (4, 11)]

GP = G * P
TAPS = L * 4
NPTS = B * Q * GP
ROWS = B * Q
NW = 32
ITERS = (ROWS + NW - 1) // NW

_LEVEL_ROWS = [B * G * N * h * w for (h, w) in FEAT_SHAPES]
_LEVEL_BASE = [int(x) for x in np.cumsum([0] + _LEVEL_ROWS)[:4]]
TABLE_ROWS = int(np.sum(_LEVEL_ROWS))

_PERM_OFF = np.zeros((G * P * 3,), np.int32)
for _g in range(G):
    for _p in range(P):
        for _c in range(3):
            _PERM_OFF[_c * 32 + _g * 8 + _p] = _g * 24 + _p * 3 + _c
_PERM_SW = np.zeros((G * P * L,), np.int32)
for _g in range(G):
    for _p in range(P):
        for _l in range(L):
            _PERM_SW[_l * 32 + _g * 8 + _p] = _g * 32 + _p * 4 + _l

_PERM_CH = np.zeros((C,), np.int32)
for _h in range(2):
    for _i in range(16):
        for _k in range(2):
            _PERM_CH[_h * 32 + 2 * _i + _k] = _h * 32 + 16 * _k + _i

_SPREAD_MASK = [2 ** int(np.floor(np.log2(B * G * N * h * w))) - 1
                for (h, w) in FEAT_SHAPES]


def _tc_body(q_ref, spc_ref, ws_ref, bs_ref,
             idx_ref, w_ref, uv_ref):
    b = pl.program_id(0)
    q = q_ref[0]
    swl = jnp.dot(q, ws_ref[...], preferred_element_type=jnp.float32) + bs_ref[0]

    s = [swl[:, l * 32:(l + 1) * 32] for l in range(L)]
    m = jnp.maximum(jnp.maximum(s[0], s[1]), jnp.maximum(s[2], s[3]))
    e = [jnp.exp(sl - m) for sl in s]
    den = e[0] + e[1] + e[2] + e[3]
    sw = [el / den for el in e]

    best_v = jnp.full((Q, GP), -1.0, jnp.float32)
    best_i = jnp.zeros((Q, GP), jnp.int32)
    u_sel = jnp.zeros((Q, GP), jnp.float32)
    v_sel = jnp.zeros((Q, GP), jnp.float32)
    for n in range(N):
        sx = spc_ref[0, n * 4 + 0]
        sy = spc_ref[0, n * 4 + 1]
        hom = spc_ref[0, n * 4 + 2]
        homnz = jnp.maximum(hom, EPS)
        un = (sx / homnz) / IMG_W
        vn = (sy / homnz) / IMG_H
        val = ((hom > EPS) & (vn > 0.0) & (vn < 1.0)
               & (un > 0.0) & (un < 1.0)).astype(jnp.float32)
        upd = val > best_v
        best_i = jnp.where(upd, n, best_i)
        u_sel = jnp.where(upd, un, u_sel)
        v_sel = jnp.where(upd, vn, v_sel)
        best_v = jnp.maximum(best_v, val)

    uv_ref[0, :, 0:32] = u_sel
    uv_ref[0, :, 32:64] = v_sel

    gcol = lax.broadcasted_iota(jnp.int32, (Q, GP), 1) // P
    bg6 = (b * G + gcol) * N + best_i
    spread = (lax.broadcasted_iota(jnp.int32, (Q, GP), 0) * GP
              + lax.broadcasted_iota(jnp.int32, (Q, GP), 1))

    for l, (H, W) in enumerate(FEAT_SHAPES):
        xl = u_sel * W - 0.5
        yl = v_sel * H - 0.5
        x0f = jnp.floor(xl)
        y0f = jnp.floor(yl)
        x0 = x0f.astype(jnp.int32)
        y0 = y0f.astype(jnp.int32)
        wx = xl - x0f
        wy = yl - y0f
        rowbase = bg6 * (H * W)
        corners = [
            (0, 0, (1.0 - wx) * (1.0 - wy)),
            (0, 1, wx * (1.0 - wy)),
            (1, 0, (1.0 - wx) * wy),
            (1, 1, wx * wy),
        ]
        for ci, (dy, dx, cw) in enumerate(corners):
            xx = x0 + dx
            yy = y0 + dy
            ok = ((xx >= 0) & (xx < W) & (yy >= 0) & (yy < H)).astype(jnp.float32)
            xc = jnp.clip(xx, 0, W - 1)
            yc = jnp.clip(yy, 0, H - 1)
            t = l * 4 + ci
            wt = cw * ok * sw[l]
            row = rowbase + yc * W + xc
            row = jnp.where(wt > 0.0,
                            row, (spread * TAPS + t) & _SPREAD_MASK[l])
            idx_ref[0, :, t * 32:(t + 1) * 32] = row
            w_ref[0, :, t * 32:(t + 1) * 32] = wt


def _tc_precompute(query, spc, ws, bs):
    return pl.pallas_call(
        _tc_body,
        grid=(B,),
        in_specs=[
            pl.BlockSpec((1, Q, D), lambda b: (b, 0, 0)),
            pl.BlockSpec((1, 4 * N, Q, GP), lambda b: (b, 0, 0, 0)),
            pl.BlockSpec((D, 128), lambda b: (0, 0)),
            pl.BlockSpec((1, 128), lambda b: (0, 0)),
        ],
        out_specs=[
            pl.BlockSpec((1, Q, 512), lambda b: (b, 0, 0)),
            pl.BlockSpec((1, Q, 512), lambda b: (b, 0, 0)),
            pl.BlockSpec((1, Q, 64), lambda b: (b, 0, 0)),
        ],
        out_shape=[
            jax.ShapeDtypeStruct((B, Q, 512), jnp.int32),
            jax.ShapeDtypeStruct((B, Q, 512), jnp.float32),
            jax.ShapeDtypeStruct((B, Q, 64), jnp.float32),
        ],
    )(query, spc, ws, bs)


def _sc_gather(tables, idx, w):
    mesh = plsc.VectorSubcoreMesh(core_axis_name="c", subcore_axis_name="s")
    RPW = ITERS

    @functools.partial(
        pl.kernel, mesh=mesh,
        out_type=jax.ShapeDtypeStruct((NPTS // 2, 2 * C), jnp.float32),
        compiler_params=pltpu.CompilerParams(use_tc_tiling_on_sc=False,
                                             needs_layout_passes=False),
        scratch_types=[
            pltpu.VMEM((4 * RPW, 128), jnp.int32),
            pltpu.VMEM((4 * RPW, 128), jnp.float32),
            pltpu.VMEM((2, 512, C), jnp.bfloat16),
            pltpu.VMEM((2, GP // 2, 2 * C), jnp.float32),
            pltpu.SemaphoreType.DMA,
            pltpu.SemaphoreType.DMA,
            pltpu.SemaphoreType.DMA,
            pltpu.SemaphoreType.DMA,
        ],
    )
    def k(t0, t1, t2, t3, idx_hbm, w_hbm, out_hbm,
          idx_v, w_v, rows_v, out_v, g0, g1, so0, so1):
        tabs = (t0, t1, t2, t3)
        gsem = (g0, g1)
        osem = (so0, so1)
        wid = lax.axis_index("s") * 2 + lax.axis_index("c")
        nrows = jnp.where(wid < 8, RPW, RPW - 1)
        base = 56 * wid + jnp.minimum(wid, 8)

        @pl.when(wid < 8)
        def _():
            pltpu.sync_copy(idx_hbm.at[pl.ds(4 * base, 4 * RPW)], idx_v)
            pltpu.sync_copy(w_hbm.at[pl.ds(4 * base, 4 * RPW)], w_v)

        @pl.when(wid >= 8)
        def _():
            pltpu.sync_copy(idx_hbm.at[pl.ds(4 * base, 4 * (RPW - 1))],
                            idx_v.at[pl.ds(0, 4 * (RPW - 1))])
            pltpu.sync_copy(w_hbm.at[pl.ds(4 * base, 4 * (RPW - 1))],
                            w_v.at[pl.ds(0, 4 * (RPW - 1))])

        def fire(j, buf):
            @pl.when(j < nrows)
            def _():
                for kk in range(4):
                    pltpu.async_copy(
                        tabs[kk].at[idx_v.at[4 * j + kk]],
                        rows_v.at[buf].at[pl.ds(kk * 128, 128)], gsem[buf])

        def consume(j, buf):
            @pl.when(j < nrows)
            def _():
                pltpu.make_async_copy(
                    t0.at[pl.ds(0, 512)], rows_v.at[buf], gsem[buf]).wait()

                @pl.when(j >= 2)
                def _():
                    pltpu.make_async_copy(
                        out_v.at[buf], out_hbm.at[pl.ds(0, GP // 2)],
                        osem[buf]).wait()

                def pbody(p, c2):
                    ti = lax.iota(jnp.int32, TAPS)
                    wvec = plsc.load_gather(
                        w_v, [4 * j + (ti >> 2), (ti & 3) * 32 + p])
                    wts = [wvec[t] for t in range(TAPS)]
                    terms = [[], [], [], []]
                    for t in range(TAPS):
                        for h in range(2):
                            xb = rows_v[buf, t * 32 + p, pl.ds(h * 32, 32)]
                            a0, a1 = plsc.unpack(
                                xb, format=plsc.PackFormat.INTERLEAVED)
                            terms[2 * h].append(a0 * wts[t])
                            terms[2 * h + 1].append(a1 * wts[t])
                    for c in range(4):
                        ts = terms[c]
                        while len(ts) > 1:
                            ts = [ts[a] + ts[a + 1]
                                  for a in range(0, len(ts), 2)]
                        out_v[buf, p >> 1,
                              pl.ds((p & 1) * C + c * 16, 16)] = ts[0]
                    return c2

                lax.fori_loop(0, GP, pbody, 0)
                pltpu.async_copy(out_v.at[buf],
                                 out_hbm.at[pl.ds((base + j) * (GP // 2),
                                                  GP // 2)],
                                 osem[buf])

        fire(0, 0)

        def body(i2, carry):
            j0 = 2 * i2
            fire(j0 + 1, 1)
            consume(j0, 0)
            fire(j0 + 2, 0)
            consume(j0 + 1, 1)
            return carry

        lax.fori_loop(0, (RPW + 1) // 2, body, 0)

        for buf in range(2):
            pltpu.make_async_copy(
                out_v.at[buf], out_hbm.at[pl.ds(0, GP // 2)],
                osem[buf]).wait()

    return k(tables[0], tables[1], tables[2], tables[3], idx, w)


def kernel(query, reference_points, lidar2img, feat0, feat1, feat2, feat3,
           W_off, b_off, W_sw, b_sw):
    perm_sw = jnp.asarray(_PERM_SW)
    ws = W_sw[:, perm_sw]
    bs = b_sw[perm_sw][None]

    so = (query @ W_off + b_off).reshape(B, Q, G, P, 3)
    rp = reference_points[:, :, None, None, :]
    rx = rp[..., 0:1] * (PC[3] - PC[0]) + PC[0]
    ry = rp[..., 1:2] * (PC[4] - PC[1]) + PC[1]
    rz = rp[..., 2:3] * (PC[5] - PC[2]) + PC[2]
    ref = jnp.concatenate([rx, ry, rz], axis=-1) + so
    sp = ref.reshape(B, Q, GP, 3)
    ones = jnp.ones_like(sp[..., :1])
    sph = jnp.concatenate([sp, ones], axis=-1)
    spc = jnp.einsum('bnij,bqpj->bniqp', lidar2img, sph)
    spc_k = spc.reshape(B, 4 * N, Q, GP)

    idx, w, uv = _tc_precompute(query, spc_k, ws, bs)

    perm_ch = jnp.asarray(_PERM_CH)
    tables = [jnp.transpose(f, (0, 2, 3, 4, 1)).reshape(-1, C)
              .astype(jnp.bfloat16)[:, perm_ch]
              for f in (feat0, feat1, feat2, feat3)]

    out = _sc_gather(tables, idx.reshape(4 * ROWS, 128),
                     w.reshape(4 * ROWS, 128))
    final = out.reshape(B, Q, G, P, C)

    u = uv[:, :, 0:32].reshape(B, Q, G, P)
    v = uv[:, :, 32:64].reshape(B, Q, G, P)
    ref2d = jnp.stack([u, v], axis=-1)
    ref2d = jnp.transpose(ref2d, (0, 2, 1, 3, 4)).reshape(B * G, Q, P, 2)
    return final, ref2d
